# Initial kernel scaffold; baseline (speedup 1.0000x reference)
import jax
import jax.numpy as jnp
from jax.experimental import pallas as pl


def kernel(psi):
    raise NotImplementedError("write your pallas kernel here")


# speedup vs baseline: 252.9911x; 252.9911x over previous
import jax
import jax.numpy as jnp
from jax.experimental import pallas as pl

_N = 16777216
_LANES = 8192
_ROWS = _N // _LANES
_BLK_ROWS = 128
_NBLK = _ROWS // _BLK_ROWS


def _body(psi_ref, out_ref):
    out_ref[0] = psi_ref[...]
    out_ref[1] = jnp.zeros_like(psi_ref)


def kernel(psi):
    x = psi.reshape(_ROWS, _LANES)
    out = pl.pallas_call(
        _body,
        grid=(_NBLK,),
        in_specs=[pl.BlockSpec((_BLK_ROWS, _LANES), lambda i: (i, 0))],
        out_specs=pl.BlockSpec((2, _BLK_ROWS, _LANES), lambda i: (0, i, 0)),
        out_shape=jax.ShapeDtypeStruct((2, _ROWS, _LANES), jnp.float32),
    )(x)
    return out.reshape(2 * _N)
